# baseline (speedup 1.0000x reference)
import functools

import jax
import jax.numpy as jnp
from jax import lax
from jax.experimental import pallas as pl
from jax.experimental.pallas import tpu as pltpu
from jax.experimental.pallas import tpu_sc as plsc


def _make_lookup(B, L, V, D, interpret=False):
    NC, NS = 2, 16
    NW = NC * NS
    assert B % NW == 0
    seq_per_w = B // NW
    mesh = plsc.VectorSubcoreMesh(core_axis_name="c", subcore_axis_name="s",
                                  num_cores=NC, num_subcores=NS)

    @functools.partial(
        pl.kernel,
        out_type=jax.ShapeDtypeStruct((B * L, D), jnp.float32),
        mesh=mesh,
        scratch_types=[
            pltpu.VMEM((L,), jnp.int32),
            pltpu.VMEM((L,), jnp.int32),
            pltpu.VMEM((L, D), jnp.float32),
            pltpu.VMEM((L, D), jnp.float32),
            pltpu.VMEM((L, D), jnp.float32),
            pltpu.SemaphoreType.DMA,
            pltpu.SemaphoreType.DMA,
            pltpu.SemaphoreType.DMA,
            pltpu.SemaphoreType.DMA,
        ],
        interpret=interpret,
        name="wte_wpe_lookup",
        compiler_params=pltpu.CompilerParams(use_tc_tiling_on_sc=False),
    )
    def lookup(ids_hbm, wte_hbm, wpe_hbm, out_hbm,
               idx_0, idx_1, rows_0, rows_1, wpe_v,
               sem_0, sem_1, osem_0, osem_1):
        wid = lax.axis_index("s") * NC + lax.axis_index("c")
        base_seq = wid * seq_per_w

        pltpu.sync_copy(wpe_hbm, wpe_v)

        idx_bufs = (idx_0, idx_1)
        row_bufs = (rows_0, rows_1)
        sems = (sem_0, sem_1)
        osems = (osem_0, osem_1)

        def fire(i, slot):
            base = (base_seq + i) * L
            pltpu.sync_copy(ids_hbm.at[pl.ds(base, L)], idx_bufs[slot])
            return pltpu.async_copy(wte_hbm.at[idx_bufs[slot]],
                                    row_bufs[slot], sems[slot])

        pending = fire(0, 0)
        out_pending = [None, None]
        for i in range(seq_per_w):
            slot = i % 2
            nxt = None
            if i + 1 < seq_per_w:
                if out_pending[1 - slot] is not None:
                    out_pending[1 - slot].wait()
                    out_pending[1 - slot] = None
                nxt = fire(i + 1, 1 - slot)
            pending.wait()
            rows_v = row_bufs[slot]

            def row_body(r, c2):
                for j in range(D // 16):
                    sl = pl.ds(j * 16, 16)
                    rows_v[r, sl] = rows_v[r, sl] + wpe_v[r, sl]
                return c2

            lax.fori_loop(0, L, row_body, 0, unroll=2)
            base = (base_seq + i) * L
            out_pending[slot] = pltpu.async_copy(
                rows_v, out_hbm.at[pl.ds(base, L)], osems[slot])
            pending = nxt
        for cp in out_pending:
            if cp is not None:
                cp.wait()

    return lookup


def kernel(input_ids, wte_table, wpe_table):
    B, L = input_ids.shape
    V, D = wte_table.shape
    ids_flat = input_ids.reshape(B * L).astype(jnp.int32)
    wpe = wpe_table[:L]
    out = _make_lookup(B, L, V, D)(ids_flat, wte_table, wpe)
    return out.reshape(B, L, D)


# speedup vs baseline: 1.1042x; 1.1042x over previous
import functools

import jax
import jax.numpy as jnp
from jax import lax
from jax.experimental import pallas as pl
from jax.experimental.pallas import tpu as pltpu
from jax.experimental.pallas import tpu_sc as plsc


def _make_lookup(B, L, V, D, interpret=False):
    NC, NS = 2, 16
    NW = NC * NS
    assert B % NW == 0
    seq_per_w = B // NW
    mesh = plsc.VectorSubcoreMesh(core_axis_name="c", subcore_axis_name="s",
                                  num_cores=NC, num_subcores=NS)

    @functools.partial(
        pl.kernel,
        out_type=jax.ShapeDtypeStruct((B * L, D), jnp.float32),
        mesh=mesh,
        scratch_types=[
            pltpu.VMEM((L,), jnp.int32),
            pltpu.VMEM((L,), jnp.int32),
            pltpu.VMEM((L, D), jnp.float32),
            pltpu.VMEM((L, D), jnp.float32),
            pltpu.VMEM((L, D), jnp.float32),
            pltpu.SemaphoreType.DMA,
            pltpu.SemaphoreType.DMA,
        ],
        interpret=interpret,
        name="wte_wpe_lookup",
        compiler_params=pltpu.CompilerParams(use_tc_tiling_on_sc=False),
    )
    def lookup(ids_hbm, wte_hbm, wpe_hbm, out_hbm,
               idx_0, idx_1, rows_0, rows_1, wpe_v, sem_0, sem_1):
        wid = lax.axis_index("s") * NC + lax.axis_index("c")
        base_seq = wid * seq_per_w

        pltpu.sync_copy(wpe_hbm, wpe_v)

        idx_bufs = (idx_0, idx_1)
        row_bufs = (rows_0, rows_1)
        sems = (sem_0, sem_1)

        def fire(i, slot):
            base = (base_seq + i) * L
            pltpu.sync_copy(ids_hbm.at[pl.ds(base, L)], idx_bufs[slot])
            return pltpu.async_copy(wte_hbm.at[idx_bufs[slot]],
                                    row_bufs[slot], sems[slot])

        pending = fire(0, 0)
        for i in range(seq_per_w):
            slot = i % 2
            nxt = fire(i + 1, 1 - slot) if i + 1 < seq_per_w else None
            pending.wait()
            rows_v = row_bufs[slot]

            def row_body(r, c2):
                for j in range(D // 16):
                    sl = pl.ds(j * 16, 16)
                    rows_v[r, sl] = rows_v[r, sl] + wpe_v[r, sl]
                return c2

            lax.fori_loop(0, L, row_body, 0)
            base = (base_seq + i) * L
            pltpu.sync_copy(rows_v, out_hbm.at[pl.ds(base, L)])
            pending = nxt

    return lookup


def kernel(input_ids, wte_table, wpe_table):
    B, L = input_ids.shape
    V, D = wte_table.shape
    ids_flat = input_ids.reshape(B * L).astype(jnp.int32)
    wpe = wpe_table[:L]
    out = _make_lookup(B, L, V, D)(ids_flat, wte_table, wpe)
    return out.reshape(B, L, D)
